# baseline (speedup 1.0000x reference)
import functools

import jax
import jax.numpy as jnp
from jax.experimental import pallas as pl
from jax.experimental.pallas import tpu as pltpu

_N = 20000
_C = 18
_NMS_PRE = 1000
_PAD = 1024
_IOU_THR = 0.5
_SCORE_THR = 0.05
_BLK = 128
_NBLK = _PAD // _BLK

_INTERPRET = False


def _smat_body(boxes_r_ref, boxes_c_ref, rank_ref, s_out_ref,
               p_ref, sbr_ref, sbc_ref):
    c = pl.program_id(0)

    rank_v = rank_ref[c]

    for rb in range(4):
        ri = jax.lax.broadcasted_iota(jnp.int32, (256, _PAD), 0) + rb * 256
        p_ref[rb * 256:(rb + 1) * 256, :] = (ri == rank_v).astype(jnp.float32)

    sbr_ref[...] = jnp.dot(p_ref[...], boxes_r_ref[...],
                           preferred_element_type=jnp.float32)
    sbc_ref[...] = jax.lax.dot_general(boxes_c_ref[...], p_ref[...],
                                       (((1,), (1,)), ((), ())),
                                       preferred_element_type=jnp.float32)

    br = sbr_ref[...]
    bc = sbc_ref[...]
    volr = ((br[:, 3:4] - br[:, 0:1]) * (br[:, 4:5] - br[:, 1:2])
            * (br[:, 5:6] - br[:, 2:3]))
    volc = ((bc[3:4, :] - bc[0:1, :]) * (bc[4:5, :] - bc[1:2, :])
            * (bc[5:6, :] - bc[2:3, :]))
    for rb in range(4):
        sl = slice(rb * 256, (rb + 1) * 256)
        inter = None
        for d0 in range(3):
            il = jnp.maximum(br[sl, d0:d0 + 1], bc[d0:d0 + 1, :])
            ih = jnp.minimum(br[sl, d0 + 3:d0 + 4], bc[d0 + 3:d0 + 4, :])
            w = jnp.clip(ih - il, 0.0, None)
            inter = w if inter is None else inter * w
        union = volr[sl, :] + volc - inter
        iou = inter / jnp.maximum(union, 1e-8)
        s_out_ref[0, sl, :] = (iou > _IOU_THR).astype(jnp.int8)


def _greedy_body(sall_ref, score_ref, keep_out_ref, keep_ref):
    score3 = score_ref[...]
    nvalid = jnp.sum((score3 > _SCORE_THR).astype(jnp.float32),
                     axis=2, keepdims=True)
    col3 = jax.lax.broadcasted_iota(jnp.int32, (1, _C, _PAD), 2)
    keep_ref[...] = jnp.where(col3.astype(jnp.float32) < nvalid, 1.0, 0.0)

    lane3 = jax.lax.broadcasted_iota(jnp.int32, (1, _C, _BLK), 2)

    def block_body(b, carry):
        base = pl.multiple_of(b * _BLK, _BLK)
        kb = keep_ref[:, :, pl.ds(base, _BLK)]
        acc = jnp.zeros((1, _C, _PAD), jnp.bfloat16)
        for i in range(_BLK):
            row3 = sall_ref[pl.ds(base + i, 1)].astype(jnp.bfloat16)
            rowb = sall_ref[pl.ds(base + i, 1), :,
                            pl.ds(base, _BLK)].astype(jnp.bfloat16)
            kti = jax.lax.slice(kb, (0, 0, i), (1, _C, i + 1)) > 0.5
            sup = (rowb > 0.5) & kti & (lane3 > i)
            kb = jnp.where(sup, 0.0, kb)
            acc = jnp.maximum(acc, jnp.where(kti, row3, jnp.bfloat16(0)))
        kf = keep_ref[...]
        kill = (acc > 0.5) & (col3 >= base + _BLK)
        keep_ref[...] = jnp.where(kill, 0.0, kf)
        keep_ref[:, :, pl.ds(base, _BLK)] = kb
        return carry

    jax.lax.fori_loop(0, _NBLK, block_body, 0)
    keep_out_ref[...] = keep_ref[...]


@jax.jit
def _nms_pallas(boxes_r, boxes_c, rank3, score3):
    s_cmats = pl.pallas_call(
        _smat_body,
        grid=(_C,),
        in_specs=[
            pl.BlockSpec((_PAD, 8), lambda c: (0, 0)),
            pl.BlockSpec((8, _PAD), lambda c: (0, 0)),
            pl.BlockSpec((_C, 1, _PAD), lambda c: (0, 0, 0)),
        ],
        out_specs=pl.BlockSpec((1, _PAD, _PAD), lambda c: (c, 0, 0)),
        out_shape=jax.ShapeDtypeStruct((_C, _PAD, _PAD), jnp.int8),
        scratch_shapes=[
            pltpu.VMEM((_PAD, _PAD), jnp.float32),
            pltpu.VMEM((_PAD, 8), jnp.float32),
            pltpu.VMEM((8, _PAD), jnp.float32),
        ],
        interpret=_INTERPRET,
    )(boxes_r, boxes_c, rank3)

    sall = jnp.transpose(s_cmats, (1, 0, 2))

    keep_s = pl.pallas_call(
        _greedy_body,
        in_specs=[
            pl.BlockSpec((_PAD, _C, _PAD), lambda: (0, 0, 0)),
            pl.BlockSpec((1, _C, _PAD), lambda: (0, 0, 0)),
        ],
        out_specs=pl.BlockSpec((1, _C, _PAD), lambda: (0, 0, 0)),
        out_shape=jax.ShapeDtypeStruct((1, _C, _PAD), jnp.float32),
        scratch_shapes=[pltpu.VMEM((1, _C, _PAD), jnp.float32)],
        interpret=_INTERPRET,
    )(sall, score3)
    return keep_s


def kernel(points, bbox_pred, cls_score):
    scores_full = jax.nn.sigmoid(cls_score)
    max_scores = jnp.max(scores_full, axis=1)
    _, ids = jax.lax.top_k(max_scores, _NMS_PRE)
    p = points[ids]
    bp = bbox_pred[ids]
    s = scores_full[ids]
    d = jnp.exp(bp)
    lo = p - d[:, :3]
    hi = p + d[:, 3:]
    boxes = jnp.concatenate([lo, hi], axis=1)

    npad = _PAD - _NMS_PRE
    s_pad = jnp.concatenate(
        [s, jnp.full((npad, _C), -1.0, jnp.float32)], axis=0)
    boxes_pad = jnp.concatenate(
        [boxes, jnp.zeros((npad, 6), jnp.float32)], axis=0)
    order = jnp.argsort(-s_pad, axis=0)
    rank = jnp.argsort(order, axis=0)
    s_sorted = -jnp.sort(-s_pad, axis=0)

    boxes_r = jnp.concatenate(
        [boxes_pad, jnp.zeros((_PAD, 2), jnp.float32)], axis=1)
    boxes_c = boxes_r.T
    rank3 = rank.T.reshape(_C, 1, _PAD).astype(jnp.int32)
    score3 = s_sorted.T.reshape(1, _C, _PAD)

    keep_s = _nms_pallas(boxes_r, boxes_c, rank3, score3)
    keep_orig = jnp.take_along_axis(keep_s[0], rank.T, axis=1)
    keepb = keep_orig[:, :_NMS_PRE].T > 0.5
    nms_scores = jnp.where(keepb, s, 0.0)
    return jnp.concatenate([boxes, nms_scores], axis=1)


# speedup vs baseline: 1.0671x; 1.0671x over previous
import functools

import jax
import jax.numpy as jnp
from jax.experimental import pallas as pl
from jax.experimental.pallas import tpu as pltpu

_N = 20000
_C = 18
_NMS_PRE = 1000
_PAD = 1024
_IOU_THR = 0.5
_SCORE_THR = 0.05
_BLK = 128
_NBLK = _PAD // _BLK

_INTERPRET = False


def _smat_body(boxes_r_ref, boxes_c_ref, rank_ref, s_out_ref,
               p_ref, sbr_ref, sbc_ref):
    c = pl.program_id(0)

    rank_v = rank_ref[c]

    for rb in range(4):
        ri = jax.lax.broadcasted_iota(jnp.int32, (256, _PAD), 0) + rb * 256
        p_ref[rb * 256:(rb + 1) * 256, :] = (ri == rank_v).astype(jnp.float32)

    sbr_ref[...] = jnp.dot(p_ref[...], boxes_r_ref[...],
                           preferred_element_type=jnp.float32)
    sbc_ref[...] = jax.lax.dot_general(boxes_c_ref[...], p_ref[...],
                                       (((1,), (1,)), ((), ())),
                                       preferred_element_type=jnp.float32)

    br = sbr_ref[...]
    bc = sbc_ref[...]
    volr = ((br[:, 3:4] - br[:, 0:1]) * (br[:, 4:5] - br[:, 1:2])
            * (br[:, 5:6] - br[:, 2:3]))
    volc = ((bc[3:4, :] - bc[0:1, :]) * (bc[4:5, :] - bc[1:2, :])
            * (bc[5:6, :] - bc[2:3, :]))
    for rb in range(4):
        sl = slice(rb * 256, (rb + 1) * 256)
        cs = slice(rb * 256, _PAD)
        inter = None
        for d0 in range(3):
            il = jnp.maximum(br[sl, d0:d0 + 1], bc[d0:d0 + 1, cs])
            ih = jnp.minimum(br[sl, d0 + 3:d0 + 4], bc[d0 + 3:d0 + 4, cs])
            w = jnp.clip(ih - il, 0.0, None)
            inter = w if inter is None else inter * w
        union = volr[sl, :] + volc[:, cs] - inter
        s_out_ref[0, sl, cs] = (
            inter > _IOU_THR * jnp.maximum(union, 1e-8)).astype(jnp.int8)


def _greedy_body(sall_ref, score_ref, keep_out_ref, keep_ref):
    score3 = score_ref[...]
    nvalid = jnp.sum((score3 > _SCORE_THR).astype(jnp.float32),
                     axis=2, keepdims=True)
    col3 = jax.lax.broadcasted_iota(jnp.int32, (1, _C, _PAD), 2)
    keep_ref[...] = jnp.where(col3.astype(jnp.float32) < nvalid, 1.0, 0.0)

    lane3 = jax.lax.broadcasted_iota(jnp.int32, (1, _C, _BLK), 2)

    def block_body(b, carry):
        base = pl.multiple_of(b * _BLK, _BLK)
        kb = keep_ref[:, :, pl.ds(base, _BLK)]
        acc = jnp.zeros((1, _C, _PAD), jnp.bfloat16)
        half = jnp.bfloat16(0.5)
        for g in range(0, _BLK, 4):
            rows = [sall_ref[pl.ds(base + g + j, 1)].astype(jnp.bfloat16)
                    for j in range(4)]
            rowbs = [sall_ref[pl.ds(base + g + j, 1), :,
                              pl.ds(base, _BLK)].astype(jnp.bfloat16)
                     for j in range(4)]
            k4 = jax.lax.slice(kb, (0, 0, g), (1, _C, g + 4)) > 0.5

            def _bit(j, m):
                return jax.lax.slice(
                    rowbs[j], (0, 0, g + m), (1, _C, g + m + 1)) > half

            k0 = jax.lax.slice(k4, (0, 0, 0), (1, _C, 1))
            k1 = jax.lax.slice(k4, (0, 0, 1), (1, _C, 2))
            k2 = jax.lax.slice(k4, (0, 0, 2), (1, _C, 3))
            k3 = jax.lax.slice(k4, (0, 0, 3), (1, _C, 4))
            k1 = k1 & ~(_bit(0, 1) & k0)
            k2 = k2 & ~(_bit(0, 2) & k0) & ~(_bit(1, 2) & k1)
            k3 = (k3 & ~(_bit(0, 3) & k0) & ~(_bit(1, 3) & k1)
                  & ~(_bit(2, 3) & k2))
            kq = (k0, k1, k2, k3)
            sup = None
            for j in range(4):
                s_j = (rowbs[j] > half) & kq[j] & (lane3 > g + j)
                sup = s_j if sup is None else sup | s_j
            kb = jnp.where(sup, 0.0, kb)
            a01 = jnp.maximum(jnp.where(kq[0], rows[0], jnp.bfloat16(0)),
                              jnp.where(kq[1], rows[1], jnp.bfloat16(0)))
            a23 = jnp.maximum(jnp.where(kq[2], rows[2], jnp.bfloat16(0)),
                              jnp.where(kq[3], rows[3], jnp.bfloat16(0)))
            acc = jnp.maximum(acc, jnp.maximum(a01, a23))
        kf = keep_ref[...]
        kill = (acc > 0.5) & (col3 >= base + _BLK)
        keep_ref[...] = jnp.where(kill, 0.0, kf)
        keep_ref[:, :, pl.ds(base, _BLK)] = kb
        return carry

    jax.lax.fori_loop(0, _NBLK, block_body, 0)
    keep_out_ref[...] = keep_ref[...]


@jax.jit
def _nms_pallas(boxes_r, boxes_c, rank3, score3):
    s_cmats = pl.pallas_call(
        _smat_body,
        grid=(_C,),
        in_specs=[
            pl.BlockSpec((_PAD, 8), lambda c: (0, 0)),
            pl.BlockSpec((8, _PAD), lambda c: (0, 0)),
            pl.BlockSpec((_C, 1, _PAD), lambda c: (0, 0, 0)),
        ],
        out_specs=pl.BlockSpec((1, _PAD, _PAD), lambda c: (c, 0, 0)),
        out_shape=jax.ShapeDtypeStruct((_C, _PAD, _PAD), jnp.int8),
        scratch_shapes=[
            pltpu.VMEM((_PAD, _PAD), jnp.float32),
            pltpu.VMEM((_PAD, 8), jnp.float32),
            pltpu.VMEM((8, _PAD), jnp.float32),
        ],
        interpret=_INTERPRET,
    )(boxes_r, boxes_c, rank3)

    sall = jnp.transpose(s_cmats, (1, 0, 2))

    keep_s = pl.pallas_call(
        _greedy_body,
        in_specs=[
            pl.BlockSpec((_PAD, _C, _PAD), lambda: (0, 0, 0)),
            pl.BlockSpec((1, _C, _PAD), lambda: (0, 0, 0)),
        ],
        out_specs=pl.BlockSpec((1, _C, _PAD), lambda: (0, 0, 0)),
        out_shape=jax.ShapeDtypeStruct((1, _C, _PAD), jnp.float32),
        scratch_shapes=[pltpu.VMEM((1, _C, _PAD), jnp.float32)],
        interpret=_INTERPRET,
    )(sall, score3)
    return keep_s


def kernel(points, bbox_pred, cls_score):
    scores_full = jax.nn.sigmoid(cls_score)
    max_scores = jnp.max(scores_full, axis=1)
    _, ids = jax.lax.top_k(max_scores, _NMS_PRE)
    p = points[ids]
    bp = bbox_pred[ids]
    s = scores_full[ids]
    d = jnp.exp(bp)
    lo = p - d[:, :3]
    hi = p + d[:, 3:]
    boxes = jnp.concatenate([lo, hi], axis=1)

    npad = _PAD - _NMS_PRE
    s_pad = jnp.concatenate(
        [s, jnp.full((npad, _C), -1.0, jnp.float32)], axis=0)
    boxes_pad = jnp.concatenate(
        [boxes, jnp.zeros((npad, 6), jnp.float32)], axis=0)
    order = jnp.argsort(-s_pad, axis=0)
    rank = jnp.argsort(order, axis=0)
    s_sorted = -jnp.sort(-s_pad, axis=0)

    boxes_r = jnp.concatenate(
        [boxes_pad, jnp.zeros((_PAD, 2), jnp.float32)], axis=1)
    boxes_c = boxes_r.T
    rank3 = rank.T.reshape(_C, 1, _PAD).astype(jnp.int32)
    score3 = s_sorted.T.reshape(1, _C, _PAD)

    keep_s = _nms_pallas(boxes_r, boxes_c, rank3, score3)
    keep_orig = jnp.take_along_axis(keep_s[0], rank.T, axis=1)
    keepb = keep_orig[:, :_NMS_PRE].T > 0.5
    nms_scores = jnp.where(keepb, s, 0.0)
    return jnp.concatenate([boxes, nms_scores], axis=1)
